# baseline (speedup 1.0000x reference)
import jax
import jax.numpy as jnp
from jax import lax
from jax.experimental import pallas as pl
from jax.experimental.pallas import tpu as pltpu
from jax.experimental.pallas import tpu_sc as plsc

VOCAB = 100000
EMBD = 128
B = 4096
L = 50

NC = 2
NS = 16
NW = NC * NS

BW = B // NW
CH = 64
NCHUNK = 2 * L
NR = 10
LA = 5


def _embed_kernel(xt_hbm, table_hbm, out_hbm, idx_v, rows_v, gsem, ssem):
    wid = lax.axis_index("s") * NC + lax.axis_index("c")
    b0 = wid * BW
    pltpu.sync_copy(xt_hbm.at[:, pl.ds(b0, BW)], idx_v)

    def start_gather(g, b):
        pltpu.async_copy(
            table_hbm.at[idx_v.at[g // 2, pl.ds((g % 2) * CH, CH)]],
            rows_v.at[b], gsem.at[b],
        )

    def wait_gather(g, b):
        pltpu.make_async_copy(
            table_hbm.at[idx_v.at[g // 2, pl.ds((g % 2) * CH, CH)]],
            rows_v.at[b], gsem.at[b],
        ).wait()

    def _store_desc(g, b):
        return pltpu.make_async_copy(
            rows_v.at[b],
            out_hbm.at[g // 2, pl.ds(b0 + (g % 2) * CH, CH)],
            ssem.at[b],
        )

    def start_store(g, b):
        _store_desc(g, b).start()

    def wait_store(g, b):
        _store_desc(g, b).wait()

    for b in range(LA):
        start_gather(b, b)

    for b in range(NR):
        g = b
        wait_gather(g, b)
        start_store(g, b)
        h, hb = g + LA, (b + LA) % NR
        if h >= NR:
            wait_store(h - NR, hb)
        start_gather(h, hb)

    @pl.loop(NR, NCHUNK - NR, step=NR)
    def _pass(g0):
        for b in range(NR):
            g = g0 + b
            wait_gather(g, b)
            start_store(g, b)
            h, hb = g + LA, (b + LA) % NR
            wait_store(h - NR, hb)
            start_gather(h, hb)

    for b in range(NR):
        g = NCHUNK - NR + b
        wait_gather(g, b)
        start_store(g, b)
        h, hb = g + LA, (b + LA) % NR
        if h < NCHUNK:
            wait_store(h - NR, hb)
            start_gather(h, hb)

    for b in range(NR):
        wait_store(NCHUNK - NR + b, b)


@jax.jit
def _embed(x, table):
    xt = jnp.swapaxes(x, 0, 1)
    mesh = plsc.VectorSubcoreMesh(
        core_axis_name="c", subcore_axis_name="s", num_cores=NC,
        num_subcores=NS,
    )
    out = pl.kernel(
        _embed_kernel,
        out_type=jax.ShapeDtypeStruct((L, B, EMBD), jnp.float32),
        mesh=mesh,
        scratch_types=[
            pltpu.VMEM((L, BW), jnp.int32),
            pltpu.VMEM((NR, CH, EMBD), jnp.float32),
            pltpu.SemaphoreType.DMA((NR,)),
            pltpu.SemaphoreType.DMA((NR,)),
        ],
    )(xt, table)
    return jnp.swapaxes(out, 0, 1)


def kernel(x, table):
    return _embed(x.astype(jnp.int32), table)


# speedup vs baseline: 1.0039x; 1.0039x over previous
import jax
import jax.numpy as jnp
from jax import lax
from jax.experimental import pallas as pl
from jax.experimental.pallas import tpu as pltpu
from jax.experimental.pallas import tpu_sc as plsc

VOCAB = 100000
EMBD = 128
B = 4096
L = 50

NC = 2
NS = 16
NW = NC * NS

BW = B // NW
NR = 5
LA = 3


def _embed_kernel(xt_hbm, table_hbm, out_hbm, idx_v, rows_v, gsem, ssem):
    wid = lax.axis_index("s") * NC + lax.axis_index("c")
    b0 = wid * BW
    pltpu.sync_copy(xt_hbm.at[:, pl.ds(b0, BW)], idx_v)

    def start_gather(g, b):
        pltpu.async_copy(table_hbm.at[idx_v.at[g]], rows_v.at[b], gsem.at[b])

    def wait_gather(g, b):
        pltpu.make_async_copy(
            table_hbm.at[idx_v.at[g]], rows_v.at[b], gsem.at[b]
        ).wait()

    def _store_desc(g, b):
        return pltpu.make_async_copy(
            rows_v.at[b], out_hbm.at[g, pl.ds(b0, BW)], ssem.at[b]
        )

    def start_store(g, b):
        _store_desc(g, b).start()

    def wait_store(g, b):
        _store_desc(g, b).wait()

    for b in range(LA):
        start_gather(b, b)

    for b in range(NR):
        g = b
        wait_gather(g, b)
        start_store(g, b)
        h, hb = g + LA, (b + LA) % NR
        if h >= NR:
            wait_store(h - NR, hb)
        start_gather(h, hb)

    @pl.loop(NR, L - NR, step=NR)
    def _pass(g0):
        for b in range(NR):
            g = g0 + b
            wait_gather(g, b)
            start_store(g, b)
            h, hb = g + LA, (b + LA) % NR
            wait_store(h - NR, hb)
            start_gather(h, hb)

    for b in range(NR):
        g = L - NR + b
        wait_gather(g, b)
        start_store(g, b)
        h, hb = g + LA, (b + LA) % NR
        if h < L:
            wait_store(h - NR, hb)
            start_gather(h, hb)

    for b in range(NR):
        wait_store(L - NR + b, b)


@jax.jit
def _embed(x, table):
    xt = jnp.swapaxes(x, 0, 1)
    mesh = plsc.VectorSubcoreMesh(
        core_axis_name="c", subcore_axis_name="s", num_cores=NC,
        num_subcores=NS,
    )
    out = pl.kernel(
        _embed_kernel,
        out_type=jax.ShapeDtypeStruct((L, B, EMBD), jnp.float32),
        mesh=mesh,
        scratch_types=[
            pltpu.VMEM((L, BW), jnp.int32),
            pltpu.VMEM((NR, BW, EMBD), jnp.float32),
            pltpu.SemaphoreType.DMA((NR,)),
            pltpu.SemaphoreType.DMA((NR,)),
        ],
        compiler_params=pltpu.CompilerParams(
            disable_bounds_checks=True,
            disable_semaphore_checks=True,
            skip_device_barrier=True,
        ),
    )(xt, table)
    return jnp.swapaxes(out, 0, 1)


def kernel(x, table):
    return _embed(x.astype(jnp.int32), table)


# speedup vs baseline: 1.0052x; 1.0013x over previous
import jax
import jax.numpy as jnp
from jax import lax
from jax.experimental import pallas as pl
from jax.experimental.pallas import tpu as pltpu
from jax.experimental.pallas import tpu_sc as plsc

VOCAB = 100000
EMBD = 128
B = 4096
L = 50

NC = 2
NS = 16
NW = NC * NS

BW = B // NW
NR = 5
LA = 4


def _embed_kernel(xt_hbm, table_hbm, out_hbm, idx_v, rows_v, gsem, ssem):
    wid = lax.axis_index("s") * NC + lax.axis_index("c")
    b0 = wid * BW
    pltpu.sync_copy(xt_hbm.at[:, pl.ds(b0, BW)], idx_v)

    def start_gather(g, b):
        pltpu.async_copy(table_hbm.at[idx_v.at[g]], rows_v.at[b], gsem.at[b])

    def wait_gather(g, b):
        pltpu.make_async_copy(
            table_hbm.at[idx_v.at[g]], rows_v.at[b], gsem.at[b]
        ).wait()

    def _store_desc(g, b):
        return pltpu.make_async_copy(
            rows_v.at[b], out_hbm.at[g, pl.ds(b0, BW)], ssem.at[b]
        )

    def start_store(g, b):
        _store_desc(g, b).start()

    def wait_store(g, b):
        _store_desc(g, b).wait()

    for b in range(LA):
        start_gather(b, b)

    for b in range(NR):
        g = b
        wait_gather(g, b)
        start_store(g, b)
        h, hb = g + LA, (b + LA) % NR
        if h >= NR:
            wait_store(h - NR, hb)
        start_gather(h, hb)

    @pl.loop(NR, L - NR, step=NR)
    def _pass(g0):
        for b in range(NR):
            g = g0 + b
            wait_gather(g, b)
            start_store(g, b)
            h, hb = g + LA, (b + LA) % NR
            wait_store(h - NR, hb)
            start_gather(h, hb)

    for b in range(NR):
        g = L - NR + b
        wait_gather(g, b)
        start_store(g, b)
        h, hb = g + LA, (b + LA) % NR
        if h < L:
            wait_store(h - NR, hb)
            start_gather(h, hb)

    for b in range(NR):
        wait_store(L - NR + b, b)


@jax.jit
def _embed(x, table):
    xt = jnp.swapaxes(x, 0, 1)
    mesh = plsc.VectorSubcoreMesh(
        core_axis_name="c", subcore_axis_name="s", num_cores=NC,
        num_subcores=NS,
    )
    out = pl.kernel(
        _embed_kernel,
        out_type=jax.ShapeDtypeStruct((L, B, EMBD), jnp.float32),
        mesh=mesh,
        scratch_types=[
            pltpu.VMEM((L, BW), jnp.int32),
            pltpu.VMEM((NR, BW, EMBD), jnp.float32),
            pltpu.SemaphoreType.DMA((NR,)),
            pltpu.SemaphoreType.DMA((NR,)),
        ],
    )(xt, table)
    return jnp.swapaxes(out, 0, 1)


def kernel(x, table):
    return _embed(x.astype(jnp.int32), table)
